# baseline (speedup 1.0000x reference)
import jax
import jax.numpy as jnp
from jax import lax
from jax.experimental import pallas as pl
from jax.experimental.pallas import tpu as pltpu
from jax.experimental.pallas import tpu_sc as plsc

B = 4096
T = 150
TP = 160
SEQ = 200
D = 300
DP = 304
NC, NS = 2, 16
NW = NC * NS
BPW = B // NW
GB = 152
ZR = SEQ - GB
CB = 16
NCHUNK = BPW // CB


def _zero_rows(buf, r0):
    z = jnp.zeros((16,), jnp.float32)
    for r in (r0, r0 + 1):
        for c in range(0, DP, 16):
            buf[r, pl.ds(c, 16)] = z


def _sc_lookup(idx_hbm, zeros_hbm, table_hbm, out_hbm,
               idx_v0, idx_v1, buf0, buf1, zero_v,
               sem_i, sem_g0, sem_g1, sem_w0, sem_w1, sem_z):
    wid = lax.axis_index("s") * NC + lax.axis_index("c")
    wbase = wid * BPW
    pltpu.sync_copy(zeros_hbm, zero_v)
    idx_bufs = (idx_v0, idx_v1)
    bufs = (buf0, buf1)
    gsems = (sem_g0, sem_g1)
    wsems = (sem_w0, sem_w1)
    pltpu.sync_copy(idx_hbm.at[pl.ds(wbase * TP, CB * TP)], idx_v0)
    pltpu.async_copy(idx_hbm.at[pl.ds((wbase + CB) * TP, CB * TP)],
                     idx_v1, sem_i)

    def make_step(pv, cbase, guard_first):
        def step(jj, carry):
            for p in range(2):
                j = jj * 2 + p
                b = wbase + cbase + j
                off = j * TP
                buf, gs = bufs[p], gsems[p]
                ia = pv.at[pl.ds(off, 128)]
                ib = pv.at[pl.ds(off + 128, GB - 128)]

                def wait_prev(buf=buf, ws=wsems[p], b=b):
                    pltpu.make_async_copy(
                        buf, out_hbm.at[b, pl.ds(0, GB)], ws).wait()

                if guard_first:
                    pl.when(jj > 0)(wait_prev)
                else:
                    wait_prev()

                pltpu.async_copy(table_hbm.at[ia], buf.at[pl.ds(0, 128)], gs)
                pltpu.async_copy(table_hbm.at[ib],
                                 buf.at[pl.ds(128, GB - 128)], gs)
                pltpu.async_copy(zero_v, out_hbm.at[b, pl.ds(GB, ZR)], sem_z)

            for p in range(2):
                j = jj * 2 + p
                b = wbase + cbase + j
                off = j * TP
                buf, gs, ws = bufs[p], gsems[p], wsems[p]
                ia = pv.at[pl.ds(off, 128)]
                ib = pv.at[pl.ds(off + 128, GB - 128)]
                pltpu.make_async_copy(table_hbm.at[ia],
                                      buf.at[pl.ds(0, 128)], gs).wait()
                pltpu.make_async_copy(table_hbm.at[ib],
                                      buf.at[pl.ds(128, GB - 128)], gs).wait()
                _zero_rows(buf, T)
                pltpu.async_copy(buf, out_hbm.at[b, pl.ds(0, GB)], ws)

            def wait_z(b0=wbase + cbase + jj * 2):
                for _ in range(2):
                    pltpu.make_async_copy(
                        zero_v, out_hbm.at[b0, pl.ds(GB, ZR)], sem_z).wait()

            if guard_first:
                pl.when(jj > 0)(wait_z)
            else:
                wait_z()
            return carry

        return step

    for c in range(NCHUNK):
        pv = idx_bufs[c % 2]
        if c > 0:
            pltpu.make_async_copy(
                idx_hbm.at[pl.ds((wbase + c * CB) * TP, CB * TP)],
                pv, sem_i).wait()
            if c < NCHUNK - 1:
                pltpu.async_copy(
                    idx_hbm.at[pl.ds((wbase + (c + 1) * CB) * TP, CB * TP)],
                    idx_bufs[(c + 1) % 2], sem_i)
        lax.fori_loop(0, CB // 2, make_step(pv, c * CB, c == 0), 0)

    bl = wbase + BPW - 1
    pltpu.make_async_copy(buf0, out_hbm.at[bl, pl.ds(0, GB)], sem_w0).wait()
    pltpu.make_async_copy(buf1, out_hbm.at[bl, pl.ds(0, GB)], sem_w1).wait()
    for _ in range(2):
        pltpu.make_async_copy(zero_v, out_hbm.at[bl, pl.ds(GB, ZR)],
                              sem_z).wait()


def kernel(indices, table):
    idx32 = jnp.pad(indices.astype(jnp.int32), ((0, 0), (0, TP - T)))
    idx_flat = idx32.reshape(B * TP)
    zeros = jnp.zeros((ZR, DP), jnp.float32)
    table_p = jnp.pad(table, ((0, 0), (0, DP - D)))
    mesh = plsc.VectorSubcoreMesh(
        core_axis_name="c", subcore_axis_name="s", num_cores=NC,
        num_subcores=NS)
    k = pl.kernel(
        _sc_lookup,
        out_type=jax.ShapeDtypeStruct((B, SEQ, DP), jnp.float32),
        mesh=mesh,
        scratch_types=[
            pltpu.VMEM((CB * TP,), jnp.int32),
            pltpu.VMEM((CB * TP,), jnp.int32),
            pltpu.VMEM((GB, DP), jnp.float32),
            pltpu.VMEM((GB, DP), jnp.float32),
            pltpu.VMEM((ZR, DP), jnp.float32),
            pltpu.SemaphoreType.DMA,
            pltpu.SemaphoreType.DMA,
            pltpu.SemaphoreType.DMA,
            pltpu.SemaphoreType.DMA,
            pltpu.SemaphoreType.DMA,
            pltpu.SemaphoreType.DMA,
        ],
        compiler_params=pltpu.CompilerParams(use_tc_tiling_on_sc=False),
    )
    out_p = k(idx_flat, zeros, table_p)
    return out_p[:, :, :D]


# speedup vs baseline: 2.1524x; 2.1524x over previous
import jax
import jax.numpy as jnp
from jax import lax
from jax.experimental import pallas as pl
from jax.experimental.pallas import tpu as pltpu
from jax.experimental.pallas import tpu_sc as plsc

B = 4096
T = 150
TP = 160
SEQ = 200
D = 300
DM = 256
DT = D - DM
NC, NS = 2, 16
NW = NC * NS
NCH = 4
BPW = B // NW // NCH
RA = 80
RB = 72
ZR = SEQ - (RA + RB)
CB = 16
NCHUNK = BPW // CB


def _zero_rows(buf, r0):
    z = jnp.zeros((16,), jnp.float32)
    for r in (r0, r0 + 1):
        for c in range(0, D - 15, 16):
            buf[r, pl.ds(c, 16)] = z
        buf[r, pl.ds(D - 16, 16)] = z


def _merge_tail(buf, tbuf, nrows):
    lane = jnp.arange(16, dtype=jnp.int32)

    def row(r, carry):
        rv = jnp.full((16,), r, jnp.int32)
        for c in (0, 16, DT - 16):
            v = plsc.load_gather(tbuf, [rv, lane + c])
            plsc.store_scatter(buf, [rv, lane + (DM + c)], v)
        return carry

    lax.fori_loop(0, nrows, row, 0)


def _sc_lookup(idx_hbm, zeros_hbm, tail_hbm, table_hbm, out_hbm,
               idx_v0, idx_v1, buf_a, tbuf_a, buf_b, tbuf_b, zero_v,
               sem_i, sem_a, sem_b, sem_wa, sem_wb, sem_z):
    wid = lax.axis_index("s") * NC + lax.axis_index("c")
    wbase = wid * BPW
    pltpu.sync_copy(zeros_hbm, zero_v)
    idx_bufs = (idx_v0, idx_v1)
    pltpu.sync_copy(idx_hbm.at[pl.ds(wbase * TP, CB * TP)], idx_v0)
    pltpu.async_copy(idx_hbm.at[pl.ds((wbase + CB) * TP, CB * TP)],
                     idx_v1, sem_i)

    def make_step(pv, cbase, guard_first):
        def step(j, carry):
            b = wbase + cbase + j
            off = j * TP
            ia = pv.at[pl.ds(off, RA)]
            ib = pv.at[pl.ds(off + RA, RB)]

            def wait_prev():
                pltpu.make_async_copy(buf_a, out_hbm.at[b, pl.ds(0, RA)],
                                      sem_wa).wait()

            if guard_first:
                pl.when(j > 0)(wait_prev)
            else:
                wait_prev()

            pltpu.async_copy(table_hbm.at[ia, pl.ds(0, DM)],
                             buf_a.at[:, pl.ds(0, DM)], sem_a)
            pltpu.async_copy(tail_hbm.at[ia], tbuf_a, sem_a)

            def wait_prev_b():
                pltpu.make_async_copy(buf_b, out_hbm.at[b, pl.ds(RA, RB)],
                                      sem_wb).wait()
                pltpu.make_async_copy(zero_v,
                                      out_hbm.at[b, pl.ds(RA + RB, ZR)],
                                      sem_z).wait()

            if guard_first:
                pl.when(j > 0)(wait_prev_b)
            else:
                wait_prev_b()

            pltpu.async_copy(table_hbm.at[ib, pl.ds(0, DM)],
                             buf_b.at[:, pl.ds(0, DM)], sem_b)
            pltpu.async_copy(tail_hbm.at[ib], tbuf_b, sem_b)
            pltpu.async_copy(zero_v, out_hbm.at[b, pl.ds(RA + RB, ZR)],
                             sem_z)

            pltpu.make_async_copy(table_hbm.at[ia, pl.ds(0, DM)],
                                  buf_a.at[:, pl.ds(0, DM)], sem_a).wait()
            pltpu.make_async_copy(tail_hbm.at[ia], tbuf_a, sem_a).wait()
            _merge_tail(buf_a, tbuf_a, RA)
            pltpu.async_copy(buf_a, out_hbm.at[b, pl.ds(0, RA)], sem_wa)

            pltpu.make_async_copy(table_hbm.at[ib, pl.ds(0, DM)],
                                  buf_b.at[:, pl.ds(0, DM)], sem_b).wait()
            pltpu.make_async_copy(tail_hbm.at[ib], tbuf_b, sem_b).wait()
            _merge_tail(buf_b, tbuf_b, RB - 2)
            _zero_rows(buf_b, RB - 2)
            pltpu.async_copy(buf_b, out_hbm.at[b, pl.ds(RA, RB)], sem_wb)
            return carry

        return step

    for c in range(NCHUNK):
        pv = idx_bufs[c % 2]
        if c > 0:
            pltpu.make_async_copy(
                idx_hbm.at[pl.ds((wbase + c * CB) * TP, CB * TP)],
                pv, sem_i).wait()
            if c < NCHUNK - 1:
                pltpu.async_copy(
                    idx_hbm.at[pl.ds((wbase + (c + 1) * CB) * TP, CB * TP)],
                    idx_bufs[(c + 1) % 2], sem_i)
        lax.fori_loop(0, CB, make_step(pv, c * CB, c == 0), 0)

    bl = wbase + BPW - 1
    pltpu.make_async_copy(buf_a, out_hbm.at[bl, pl.ds(0, RA)], sem_wa).wait()
    pltpu.make_async_copy(buf_b, out_hbm.at[bl, pl.ds(RA, RB)],
                          sem_wb).wait()
    pltpu.make_async_copy(zero_v, out_hbm.at[bl, pl.ds(RA + RB, ZR)],
                          sem_z).wait()


def kernel(indices, table):
    idx32 = jnp.pad(indices.astype(jnp.int32), ((0, 0), (0, TP - T)))
    idx_chunks = idx32.reshape(NCH, (B // NCH) * TP)
    zeros = jnp.zeros((ZR, D), jnp.float32)
    tail_tbl = jnp.pad(table[:, DM:], ((0, 0), (0, 128 - DT)))
    mesh = plsc.VectorSubcoreMesh(
        core_axis_name="c", subcore_axis_name="s", num_cores=NC,
        num_subcores=NS)
    k = pl.kernel(
        _sc_lookup,
        out_type=jax.ShapeDtypeStruct((B // NCH, SEQ, D), jnp.float32),
        mesh=mesh,
        scratch_types=[
            pltpu.VMEM((CB * TP,), jnp.int32),
            pltpu.VMEM((CB * TP,), jnp.int32),
            pltpu.VMEM((RA, D), jnp.float32),
            pltpu.VMEM((RA, 128), jnp.float32),
            pltpu.VMEM((RB, D), jnp.float32),
            pltpu.VMEM((RB, 128), jnp.float32),
            pltpu.VMEM((ZR, D), jnp.float32),
            pltpu.SemaphoreType.DMA,
            pltpu.SemaphoreType.DMA,
            pltpu.SemaphoreType.DMA,
            pltpu.SemaphoreType.DMA,
            pltpu.SemaphoreType.DMA,
            pltpu.SemaphoreType.DMA,
        ],
        compiler_params=pltpu.CompilerParams(needs_layout_passes=False),
    )
    outs = [k(idx_chunks[i], zeros, tail_tbl, table) for i in range(NCH)]
    return jnp.concatenate(outs, axis=0)


# speedup vs baseline: 2.6542x; 1.2332x over previous
import jax
import jax.numpy as jnp
from jax import lax
from jax.experimental import pallas as pl
from jax.experimental.pallas import tpu as pltpu
from jax.experimental.pallas import tpu_sc as plsc

B = 4096
T = 150
TP = 160
SEQ = 200
D = 300
DM = 256
DT = D - DM
NC, NS = 2, 16
NW = NC * NS
BPW = B // NW
RA = 80
RB = 72
ZR = SEQ - (RA + RB)
CB = 16
NCHUNK = BPW // CB


def _zero_rows(buf, r0):
    z = jnp.zeros((16,), jnp.float32)
    for r in (r0, r0 + 1):
        for c in range(0, D - 15, 16):
            buf[r, pl.ds(c, 16)] = z
        buf[r, pl.ds(D - 16, 16)] = z


def _merge_tail(buf, tbuf, nrows):
    lane = jnp.arange(16, dtype=jnp.int32)

    def row(r, carry):
        rv = jnp.full((16,), r, jnp.int32)
        for cs, cd in ((84, DM), (100, DM + 16), (112, D - 16)):
            v = plsc.load_gather(tbuf, [rv, lane + cs])
            plsc.store_scatter(buf, [rv, lane + cd], v)
        return carry

    lax.fori_loop(0, nrows, row, 0)


def _sc_lookup(idx_hbm, zeros_hbm, tail_hbm, table_hbm, out_hbm,
               idx_v0, idx_v1, buf_a, tbuf_a, buf_b, tbuf_b, zero_v,
               sem_i, sem_a, sem_b, sem_wa, sem_wb, sem_z):
    wid = lax.axis_index("s") * NC + lax.axis_index("c")
    wbase = wid * BPW
    pltpu.sync_copy(zeros_hbm, zero_v)
    idx_bufs = (idx_v0, idx_v1)
    pltpu.sync_copy(idx_hbm.at[pl.ds(wbase * TP, CB * TP)], idx_v0)
    pltpu.async_copy(idx_hbm.at[pl.ds((wbase + CB) * TP, CB * TP)],
                     idx_v1, sem_i)

    def make_step(pv, cbase, guard_first):
        def step(j, carry):
            b = wbase + cbase + j
            off = j * TP
            ia = pv.at[pl.ds(off, RA)]
            ib = pv.at[pl.ds(off + RA, RB)]

            def wait_prev():
                pltpu.make_async_copy(buf_a, out_hbm.at[b, pl.ds(0, RA)],
                                      sem_wa).wait()

            if guard_first:
                pl.when(j > 0)(wait_prev)
            else:
                wait_prev()

            pltpu.async_copy(table_hbm.at[ia, pl.ds(0, DM)],
                             buf_a.at[:, pl.ds(0, DM)], sem_a)
            pltpu.async_copy(tail_hbm.at[ia], tbuf_a, sem_a)

            def wait_prev_b():
                pltpu.make_async_copy(buf_b, out_hbm.at[b, pl.ds(RA, RB)],
                                      sem_wb).wait()
                pltpu.make_async_copy(zero_v,
                                      out_hbm.at[b, pl.ds(RA + RB, ZR)],
                                      sem_z).wait()

            if guard_first:
                pl.when(j > 0)(wait_prev_b)
            else:
                wait_prev_b()

            pltpu.async_copy(table_hbm.at[ib, pl.ds(0, DM)],
                             buf_b.at[:, pl.ds(0, DM)], sem_b)
            pltpu.async_copy(tail_hbm.at[ib], tbuf_b, sem_b)
            pltpu.async_copy(zero_v, out_hbm.at[b, pl.ds(RA + RB, ZR)],
                             sem_z)

            pltpu.make_async_copy(table_hbm.at[ia, pl.ds(0, DM)],
                                  buf_a.at[:, pl.ds(0, DM)], sem_a).wait()
            pltpu.make_async_copy(tail_hbm.at[ia], tbuf_a, sem_a).wait()
            _merge_tail(buf_a, tbuf_a, RA)
            pltpu.async_copy(buf_a, out_hbm.at[b, pl.ds(0, RA)], sem_wa)

            pltpu.make_async_copy(table_hbm.at[ib, pl.ds(0, DM)],
                                  buf_b.at[:, pl.ds(0, DM)], sem_b).wait()
            pltpu.make_async_copy(tail_hbm.at[ib], tbuf_b, sem_b).wait()
            _merge_tail(buf_b, tbuf_b, RB - 2)
            _zero_rows(buf_b, RB - 2)
            pltpu.async_copy(buf_b, out_hbm.at[b, pl.ds(RA, RB)], sem_wb)
            return carry

        return step

    for c in range(NCHUNK):
        pv = idx_bufs[c % 2]
        if c > 0:
            pltpu.make_async_copy(
                idx_hbm.at[pl.ds((wbase + c * CB) * TP, CB * TP)],
                pv, sem_i).wait()
            if c < NCHUNK - 1:
                pltpu.async_copy(
                    idx_hbm.at[pl.ds((wbase + (c + 1) * CB) * TP, CB * TP)],
                    idx_bufs[(c + 1) % 2], sem_i)
        lax.fori_loop(0, CB, make_step(pv, c * CB, c == 0), 0)

    bl = wbase + BPW - 1
    pltpu.make_async_copy(buf_a, out_hbm.at[bl, pl.ds(0, RA)], sem_wa).wait()
    pltpu.make_async_copy(buf_b, out_hbm.at[bl, pl.ds(RA, RB)],
                          sem_wb).wait()
    pltpu.make_async_copy(zero_v, out_hbm.at[bl, pl.ds(RA + RB, ZR)],
                          sem_z).wait()


def kernel(indices, table):
    idx32 = jnp.pad(indices.astype(jnp.int32), ((0, 0), (0, TP - T)))
    idx_flat = idx32.reshape(B * TP)
    zeros = jnp.zeros((ZR, D), jnp.float32)
    tail_tbl = table[:, D - 128:]
    mesh = plsc.VectorSubcoreMesh(
        core_axis_name="c", subcore_axis_name="s", num_cores=NC,
        num_subcores=NS)
    k = pl.kernel(
        _sc_lookup,
        out_type=jax.ShapeDtypeStruct((B, SEQ, D), jnp.float32),
        mesh=mesh,
        scratch_types=[
            pltpu.VMEM((CB * TP,), jnp.int32),
            pltpu.VMEM((CB * TP,), jnp.int32),
            pltpu.VMEM((RA, D), jnp.float32),
            pltpu.VMEM((RA, 128), jnp.float32),
            pltpu.VMEM((RB, D), jnp.float32),
            pltpu.VMEM((RB, 128), jnp.float32),
            pltpu.VMEM((ZR, D), jnp.float32),
            pltpu.SemaphoreType.DMA,
            pltpu.SemaphoreType.DMA,
            pltpu.SemaphoreType.DMA,
            pltpu.SemaphoreType.DMA,
            pltpu.SemaphoreType.DMA,
            pltpu.SemaphoreType.DMA,
        ],
        compiler_params=pltpu.CompilerParams(needs_layout_passes=False),
    )
    return k(idx_flat, zeros, tail_tbl, table)


# speedup vs baseline: 2.6589x; 1.0017x over previous
import jax
import jax.numpy as jnp
from jax import lax
from jax.experimental import pallas as pl
from jax.experimental.pallas import tpu as pltpu
from jax.experimental.pallas import tpu_sc as plsc

B = 4096
T = 150
TP = 160
SEQ = 200
D = 300
DM = 256
DT = D - DM
NC, NS = 2, 16
NW = NC * NS
BPW = B // NW
RA = 80
RB = 72
RBZ = SEQ - RA
ZR = SEQ - (RA + RB)
CB = 16
NCHUNK = BPW // CB


def _zero_rows(buf, r0):
    z = jnp.zeros((16,), jnp.float32)
    for r in (r0, r0 + 1):
        for c in range(0, D - 15, 16):
            buf[r, pl.ds(c, 16)] = z
        buf[r, pl.ds(D - 16, 16)] = z


def _merge_tail(buf, tbuf, nrows):
    lane = jnp.arange(16, dtype=jnp.int32)

    def row(r, carry):
        rv = jnp.full((16,), r, jnp.int32)
        for cs, cd in ((84, DM), (100, DM + 16), (112, D - 16)):
            v = plsc.load_gather(tbuf, [rv, lane + cs])
            plsc.store_scatter(buf, [rv, lane + cd], v)
        return carry

    lax.fori_loop(0, nrows, row, 0)


def _sc_lookup(idx_hbm, zeros_hbm, tail_hbm, table_hbm, out_hbm,
               idx_v0, idx_v1, buf_a, tbuf_a, buf_b, tbuf_b,
               sem_i, sem_a, sem_b, sem_wa, sem_wb):
    wid = lax.axis_index("s") * NC + lax.axis_index("c")
    wbase = wid * BPW
    pltpu.sync_copy(zeros_hbm, buf_b.at[pl.ds(RB, ZR)])
    idx_bufs = (idx_v0, idx_v1)
    pltpu.sync_copy(idx_hbm.at[pl.ds(wbase * TP, CB * TP)], idx_v0)
    pltpu.async_copy(idx_hbm.at[pl.ds((wbase + CB) * TP, CB * TP)],
                     idx_v1, sem_i)

    def make_step(pv, cbase, guard_first):
        def step(j, carry):
            b = wbase + cbase + j
            off = j * TP
            ia = pv.at[pl.ds(off, RA)]
            ib = pv.at[pl.ds(off + RA, RB)]

            def wait_prev():
                pltpu.make_async_copy(buf_a, out_hbm.at[b, pl.ds(0, RA)],
                                      sem_wa).wait()

            if guard_first:
                pl.when(j > 0)(wait_prev)
            else:
                wait_prev()

            pltpu.async_copy(table_hbm.at[ia, pl.ds(0, DM)],
                             buf_a.at[:, pl.ds(0, DM)], sem_a)
            pltpu.async_copy(tail_hbm.at[ia], tbuf_a, sem_a)

            def wait_prev_b():
                pltpu.make_async_copy(buf_b, out_hbm.at[b, pl.ds(RA, RBZ)],
                                      sem_wb).wait()

            if guard_first:
                pl.when(j > 0)(wait_prev_b)
            else:
                wait_prev_b()

            pltpu.async_copy(table_hbm.at[ib, pl.ds(0, DM)],
                             buf_b.at[pl.ds(0, RB), pl.ds(0, DM)], sem_b)
            pltpu.async_copy(tail_hbm.at[ib], tbuf_b, sem_b)

            pltpu.make_async_copy(table_hbm.at[ia, pl.ds(0, DM)],
                                  buf_a.at[:, pl.ds(0, DM)], sem_a).wait()
            pltpu.make_async_copy(tail_hbm.at[ia], tbuf_a, sem_a).wait()
            _merge_tail(buf_a, tbuf_a, RA)
            pltpu.async_copy(buf_a, out_hbm.at[b, pl.ds(0, RA)], sem_wa)

            pltpu.make_async_copy(table_hbm.at[ib, pl.ds(0, DM)],
                                  buf_b.at[pl.ds(0, RB), pl.ds(0, DM)],
                                  sem_b).wait()
            pltpu.make_async_copy(tail_hbm.at[ib], tbuf_b, sem_b).wait()
            _merge_tail(buf_b, tbuf_b, RB - 2)
            _zero_rows(buf_b, RB - 2)
            pltpu.async_copy(buf_b, out_hbm.at[b, pl.ds(RA, RBZ)], sem_wb)
            return carry

        return step

    for c in range(NCHUNK):
        pv = idx_bufs[c % 2]
        if c > 0:
            pltpu.make_async_copy(
                idx_hbm.at[pl.ds((wbase + c * CB) * TP, CB * TP)],
                pv, sem_i).wait()
            if c < NCHUNK - 1:
                pltpu.async_copy(
                    idx_hbm.at[pl.ds((wbase + (c + 1) * CB) * TP, CB * TP)],
                    idx_bufs[(c + 1) % 2], sem_i)
        lax.fori_loop(0, CB, make_step(pv, c * CB, c == 0), 0)

    bl = wbase + BPW - 1
    pltpu.make_async_copy(buf_a, out_hbm.at[bl, pl.ds(0, RA)], sem_wa).wait()
    pltpu.make_async_copy(buf_b, out_hbm.at[bl, pl.ds(RA, RBZ)],
                          sem_wb).wait()


def kernel(indices, table):
    idx32 = jnp.pad(indices.astype(jnp.int32), ((0, 0), (0, TP - T)))
    idx_flat = idx32.reshape(B * TP)
    zeros = jnp.zeros((ZR, D), jnp.float32)
    tail_tbl = table[:, D - 128:]
    mesh = plsc.VectorSubcoreMesh(
        core_axis_name="c", subcore_axis_name="s", num_cores=NC,
        num_subcores=NS)
    k = pl.kernel(
        _sc_lookup,
        out_type=jax.ShapeDtypeStruct((B, SEQ, D), jnp.float32),
        mesh=mesh,
        scratch_types=[
            pltpu.VMEM((CB * TP,), jnp.int32),
            pltpu.VMEM((CB * TP,), jnp.int32),
            pltpu.VMEM((RA, D), jnp.float32),
            pltpu.VMEM((RA, 128), jnp.float32),
            pltpu.VMEM((RBZ, D), jnp.float32),
            pltpu.VMEM((RB, 128), jnp.float32),
            pltpu.SemaphoreType.DMA,
            pltpu.SemaphoreType.DMA,
            pltpu.SemaphoreType.DMA,
            pltpu.SemaphoreType.DMA,
            pltpu.SemaphoreType.DMA,
        ],
        compiler_params=pltpu.CompilerParams(needs_layout_passes=False),
    )
    return k(idx_flat, zeros, tail_tbl, table)
